# baseline (speedup 1.0000x reference)
import functools

import jax
import jax.numpy as jnp
from jax import lax
from jax.experimental import pallas as pl
from jax.experimental.pallas import tpu as pltpu
from jax.experimental.pallas import tpu_sc as plsc

VOCAB = 50257
HIDDEN = 768
MAX_POS = 8192
BATCH = 4
SEQ = 2048

NUM_CORES = 2
NUM_SUBCORES = 16
NUM_WORKERS = NUM_CORES * NUM_SUBCORES
POS_PER_WORKER = SEQ // NUM_WORKERS
PER_WORKER = BATCH * POS_PER_WORKER
TOTAL = BATCH * SEQ
CHUNK = 16
CHUNKS_PER_BATCH = POS_PER_WORKER // CHUNK
NCHUNKS = BATCH * CHUNKS_PER_BATCH
LANES = 16
VECS_PER_ROW = HIDDEN // LANES
NBUF = 6
DEPTH = 4


def _emb_body(ids_hbm, tok_hbm, pos_hbm, out_hbm,
              idx_v, pos_local, tok_bufs, gsems, psem, isems, osems):
    wid = lax.axis_index("s") * NUM_CORES + lax.axis_index("c")
    p0 = wid * POS_PER_WORKER

    pltpu.sync_copy(ids_hbm.at[pl.ds(p0, POS_PER_WORKER)], idx_v.at[0])
    ph = pltpu.async_copy(pos_hbm.at[pl.ds(p0, POS_PER_WORKER)],
                          pos_local, psem)
    ih = [None] * BATCH
    for bi in range(1, BATCH):
        ih[bi] = pltpu.async_copy(
            ids_hbm.at[pl.ds(bi * SEQ + p0, POS_PER_WORKER)],
            idx_v.at[bi], isems.at[bi])

    gh = [None] * NCHUNKS
    oh = [None] * NCHUNKS

    def start_gather(c):
        b = c % NBUF
        bi, h = divmod(c, CHUNKS_PER_BATCH)
        if bi >= 1 and h == 0:
            ih[bi].wait()
        gh[c] = pltpu.async_copy(
            tok_hbm.at[idx_v.at[bi, pl.ds(h * CHUNK, CHUNK)]],
            tok_bufs.at[b], gsems.at[b])

    for c in range(DEPTH):
        start_gather(c)
    ph.wait()

    for c in range(NCHUNKS):
        b = c % NBUF
        bi, h = divmod(c, CHUNKS_PER_BATCH)
        gh[c].wait()

        nc = c + DEPTH
        if nc < NCHUNKS:
            if nc >= NBUF:
                oh[nc - NBUF].wait()
            start_gather(nc)

        @plsc.parallel_loop(0, CHUNK)
        def add_row(r):
            for j in range(VECS_PER_ROW):
                sl = pl.ds(j * LANES, LANES)
                plsc.addupdate(tok_bufs.at[b, r, sl],
                               pos_local[h * CHUNK + r, sl])

        oh[c] = pltpu.async_copy(
            tok_bufs.at[b],
            out_hbm.at[pl.ds(bi * SEQ + p0 + h * CHUNK, CHUNK)],
            osems.at[b])

    for c in range(NCHUNKS - NBUF, NCHUNKS):
        oh[c].wait()


@jax.jit
def _emb(ids_flat, token_table, pos_table):
    mesh = plsc.VectorSubcoreMesh(core_axis_name="c", subcore_axis_name="s")
    k = functools.partial(
        pl.kernel,
        out_type=jax.ShapeDtypeStruct((TOTAL, HIDDEN), jnp.float32),
        mesh=mesh,
        scratch_types=[
            pltpu.VMEM((BATCH, POS_PER_WORKER), jnp.int32),
            pltpu.VMEM((POS_PER_WORKER, HIDDEN), jnp.float32),
            pltpu.VMEM((NBUF, CHUNK, HIDDEN), jnp.float32),
            pltpu.SemaphoreType.DMA((NBUF,)),
            pltpu.SemaphoreType.DMA,
            pltpu.SemaphoreType.DMA((BATCH,)),
            pltpu.SemaphoreType.DMA((NBUF,)),
        ],
    )(_emb_body)
    return k(ids_flat, token_table, pos_table)


def kernel(input_ids, token_table, pos_table):
    ids_flat = input_ids.reshape(-1).astype(jnp.int32)
    out = _emb(ids_flat, token_table, pos_table)
    return out.reshape(BATCH, SEQ, HIDDEN)


# speedup vs baseline: 1.0653x; 1.0653x over previous
import functools

import jax
import jax.numpy as jnp
from jax import lax
from jax.experimental import pallas as pl
from jax.experimental.pallas import tpu as pltpu
from jax.experimental.pallas import tpu_sc as plsc

VOCAB = 50257
HIDDEN = 768
MAX_POS = 8192
BATCH = 4
SEQ = 2048

NUM_CORES = 2
NUM_SUBCORES = 16
NUM_WORKERS = NUM_CORES * NUM_SUBCORES
POS_PER_WORKER = SEQ // NUM_WORKERS
TOTAL = BATCH * SEQ
CHUNKP = 8
NCHUNKS = POS_PER_WORKER // CHUNKP
LANES = 16
VECS_PER_ROW = HIDDEN // LANES
NBUF = 3
DEPTH = 2


def _emb_body(ids_hbm, tok_hbm, pos_hbm, out_hbm,
              idx_v, pos_local, tok_bufs, gsems, psem, isems, osems):
    wid = lax.axis_index("s") * NUM_CORES + lax.axis_index("c")
    p0 = wid * POS_PER_WORKER

    ih = [pltpu.async_copy(
        ids_hbm.at[pl.ds(bi * SEQ + p0, POS_PER_WORKER)],
        idx_v.at[bi], isems.at[bi]) for bi in range(BATCH)]
    ph = pltpu.async_copy(pos_hbm.at[pl.ds(p0, POS_PER_WORKER)],
                          pos_local, psem)
    for h in ih:
        h.wait()

    gh = [None] * NCHUNKS
    oh = [None] * NCHUNKS

    def start_gather(c):
        b = c % NBUF
        gh[c] = [pltpu.async_copy(
            tok_hbm.at[idx_v.at[bi, pl.ds(c * CHUNKP, CHUNKP)]],
            tok_bufs.at[b, bi], gsems.at[b]) for bi in range(BATCH)]

    for c in range(DEPTH):
        start_gather(c)
    ph.wait()

    for c in range(NCHUNKS):
        b = c % NBUF
        for h in gh[c]:
            h.wait()

        nc = c + DEPTH
        if nc < NCHUNKS:
            if nc >= NBUF:
                for h in oh[nc - NBUF]:
                    h.wait()
            start_gather(nc)

        @plsc.parallel_loop(0, CHUNKP)
        def add_row(r):
            for j in range(VECS_PER_ROW):
                sl = pl.ds(j * LANES, LANES)
                p = pos_local[c * CHUNKP + r, sl]
                for bi in range(BATCH):
                    plsc.addupdate(tok_bufs.at[b, bi, r, sl], p)

        oh[c] = [pltpu.async_copy(
            tok_bufs.at[b, bi],
            out_hbm.at[pl.ds(bi * SEQ + p0 + c * CHUNKP, CHUNKP)],
            osems.at[b]) for bi in range(BATCH)]

    for c in range(NCHUNKS - NBUF, NCHUNKS):
        for h in oh[c]:
            h.wait()


@jax.jit
def _emb(ids_flat, token_table, pos_table):
    mesh = plsc.VectorSubcoreMesh(core_axis_name="c", subcore_axis_name="s")
    k = functools.partial(
        pl.kernel,
        out_type=jax.ShapeDtypeStruct((TOTAL, HIDDEN), jnp.float32),
        mesh=mesh,
        scratch_types=[
            pltpu.VMEM((BATCH, POS_PER_WORKER), jnp.int32),
            pltpu.VMEM((POS_PER_WORKER, HIDDEN), jnp.float32),
            pltpu.VMEM((NBUF, BATCH, CHUNKP, HIDDEN), jnp.float32),
            pltpu.SemaphoreType.DMA((NBUF,)),
            pltpu.SemaphoreType.DMA,
            pltpu.SemaphoreType.DMA((BATCH,)),
            pltpu.SemaphoreType.DMA((NBUF,)),
        ],
    )(_emb_body)
    return k(ids_flat, token_table, pos_table)


def kernel(input_ids, token_table, pos_table):
    ids_flat = input_ids.reshape(-1).astype(jnp.int32)
    out = _emb(ids_flat, token_table, pos_table)
    return out.reshape(BATCH, SEQ, HIDDEN)


# speedup vs baseline: 1.0702x; 1.0046x over previous
import functools

import jax
import jax.numpy as jnp
from jax import lax
from jax.experimental import pallas as pl
from jax.experimental.pallas import tpu as pltpu
from jax.experimental.pallas import tpu_sc as plsc

VOCAB = 50257
HIDDEN = 768
MAX_POS = 8192
BATCH = 4
SEQ = 2048

NUM_CORES = 2
NUM_SUBCORES = 16
NUM_WORKERS = NUM_CORES * NUM_SUBCORES
POS_PER_WORKER = SEQ // NUM_WORKERS
TOTAL = BATCH * SEQ
CHUNKP = 8
NCHUNKS = POS_PER_WORKER // CHUNKP
ROWS = BATCH * CHUNKP
LANES = 16
VECS_PER_ROW = HIDDEN // LANES
NBUF = 3
DEPTH = 2
NBLOCKS = SEQ // CHUNKP


def _emb_body(ids_pm_hbm, tok_hbm, pos_hbm, out_hbm,
              idx_r, pos_local, tok_bufs, gsems, psem, isem, osems):
    wid = lax.axis_index("s") * NUM_CORES + lax.axis_index("c")
    p0 = wid * POS_PER_WORKER

    ih = pltpu.async_copy(ids_pm_hbm.at[pl.ds(wid * NCHUNKS, NCHUNKS)],
                          idx_r, isem)
    ph = pltpu.async_copy(pos_hbm.at[pl.ds(p0, POS_PER_WORKER)],
                          pos_local, psem)
    ih.wait()

    gh = [None] * NCHUNKS
    oh = [None] * NCHUNKS

    def start_gather(c):
        b = c % NBUF
        gh[c] = pltpu.async_copy(
            tok_hbm.at[idx_r.at[c]], tok_bufs.at[b], gsems.at[b])

    for c in range(DEPTH):
        start_gather(c)
    ph.wait()

    for c in range(NCHUNKS):
        b = c % NBUF
        gh[c].wait()

        nc = c + DEPTH
        if nc < NCHUNKS:
            if nc >= NBUF:
                for h in oh[nc - NBUF]:
                    h.wait()
            start_gather(nc)

        @plsc.parallel_loop(0, CHUNKP)
        def add_row(r):
            for j in range(VECS_PER_ROW):
                sl = pl.ds(j * LANES, LANES)
                p = pos_local[c * CHUNKP + r, sl]
                for bi in range(BATCH):
                    plsc.addupdate(tok_bufs.at[b, bi * CHUNKP + r, sl], p)

        oh[c] = [pltpu.async_copy(
            tok_bufs.at[b, pl.ds(bi * CHUNKP, CHUNKP)],
            out_hbm.at[pl.ds(bi * SEQ + p0 + c * CHUNKP, CHUNKP)],
            osems.at[b]) for bi in range(BATCH)]

    for c in range(NCHUNKS - NBUF, NCHUNKS):
        for h in oh[c]:
            h.wait()


@jax.jit
def _emb(ids_pm, token_table, pos_table):
    mesh = plsc.VectorSubcoreMesh(core_axis_name="c", subcore_axis_name="s")
    k = functools.partial(
        pl.kernel,
        out_type=jax.ShapeDtypeStruct((TOTAL, HIDDEN), jnp.float32),
        mesh=mesh,
        scratch_types=[
            pltpu.VMEM((NCHUNKS, ROWS), jnp.int32),
            pltpu.VMEM((POS_PER_WORKER, HIDDEN), jnp.float32),
            pltpu.VMEM((NBUF, ROWS, HIDDEN), jnp.float32),
            pltpu.SemaphoreType.DMA((NBUF,)),
            pltpu.SemaphoreType.DMA,
            pltpu.SemaphoreType.DMA,
            pltpu.SemaphoreType.DMA((NBUF,)),
        ],
    )(_emb_body)
    return k(ids_pm, token_table, pos_table)


def kernel(input_ids, token_table, pos_table):
    ids_pm = (input_ids.astype(jnp.int32)
              .transpose(1, 0)
              .reshape(NBLOCKS, CHUNKP, BATCH)
              .transpose(0, 2, 1)
              .reshape(NBLOCKS, ROWS))
    out = _emb(ids_pm, token_table, pos_table)
    return out.reshape(BATCH, SEQ, HIDDEN)
